# baseline (speedup 1.0000x reference)
import functools

import jax
import jax.numpy as jnp
from jax import lax
from jax.experimental import pallas as pl
from jax.experimental.pallas import tpu as pltpu
from jax.experimental.pallas import tpu_sc as plsc

N_SPEAKERS = 1000
BATCH = 16384

NUM_CORES = 2
NUM_SUBCORES = 16
LANES = 16
NUM_WORKERS = NUM_CORES * NUM_SUBCORES
ROWS_PER_WORKER = BATCH // NUM_WORKERS
CHUNK_ROWS = 64
NUM_CHUNKS = ROWS_PER_WORKER // CHUNK_ROWS
CHUNK_WORDS = CHUNK_ROWS * N_SPEAKERS
GROUPS = CHUNK_ROWS // LANES

_mesh = plsc.VectorSubcoreMesh(
    core_axis_name="c",
    subcore_axis_name="s",
    num_cores=NUM_CORES,
    num_subcores=NUM_SUBCORES,
)


@functools.partial(
    pl.kernel,
    out_type=jax.ShapeDtypeStruct((BATCH * N_SPEAKERS,), jnp.float32),
    mesh=_mesh,
    scratch_types=[
        pltpu.VMEM((ROWS_PER_WORKER,), jnp.int32),
        pltpu.VMEM((CHUNK_WORDS,), jnp.float32),
        pltpu.VMEM((CHUNK_WORDS,), jnp.float32),
        pltpu.SemaphoreType.DMA,
        pltpu.SemaphoreType.DMA,
    ],
    compiler_params=pltpu.CompilerParams(needs_layout_passes=False),
)
def _onehot_sc(ids_hbm, out_hbm, ids_v, buf0, buf1, sem0, sem1):
    wid = lax.axis_index("s") * NUM_CORES + lax.axis_index("c")
    row_base = wid * ROWS_PER_WORKER

    pltpu.sync_copy(ids_hbm.at[pl.ds(row_base, ROWS_PER_WORKER)], ids_v)

    zeros = jnp.zeros((LANES,), jnp.float32)
    ones = jnp.ones((LANES,), jnp.float32)
    lane_iota = lax.iota(jnp.int32, LANES)

    def zero_body(i, carry):
        buf0[pl.ds(i * LANES, LANES)] = zeros
        buf1[pl.ds(i * LANES, LANES)] = zeros
        return carry

    lax.fori_loop(0, CHUNK_WORDS // LANES, zero_body, 0)

    bufs = (buf0, buf1)
    sems = (sem0, sem1)
    copies = [None, None]
    prev_idx = [None, None]

    for c in range(NUM_CHUNKS):
        b = c % 2
        buf = bufs[b]
        if copies[b] is not None:
            copies[b].wait()
            for idx in prev_idx[b]:
                plsc.store_scatter(buf, [idx], zeros)
        idxs = []
        for g in range(GROUPS):
            cols = ids_v[pl.ds(c * CHUNK_ROWS + g * LANES, LANES)]
            flat = (lane_iota + g * LANES) * N_SPEAKERS + cols
            plsc.store_scatter(buf, [flat], ones)
            idxs.append(flat)
        prev_idx[b] = idxs
        dst = out_hbm.at[
            pl.ds((row_base + c * CHUNK_ROWS) * N_SPEAKERS, CHUNK_WORDS)
        ]
        copies[b] = pltpu.async_copy(buf, dst, sems[b])

    for b in range(2):
        if copies[b] is not None:
            copies[b].wait()


def kernel(style_id):
    flat = _onehot_sc(style_id)
    return flat.reshape(BATCH, N_SPEAKERS)


# speedup vs baseline: 1.7771x; 1.7771x over previous
import functools

import jax
import jax.numpy as jnp
from jax import lax
from jax.experimental import pallas as pl
from jax.experimental.pallas import tpu as pltpu
from jax.experimental.pallas import tpu_sc as plsc

N_SPEAKERS = 1000
BATCH = 16384

NUM_CORES = 2
NUM_SUBCORES = 16
LANES = 16
NUM_WORKERS = NUM_CORES * NUM_SUBCORES
ROWS_PER_WORKER = BATCH // NUM_WORKERS
CHUNK_ROWS = 32
NUM_CHUNKS = ROWS_PER_WORKER // CHUNK_ROWS
GROUPS = CHUNK_ROWS // LANES

_COL_OFFSETS = tuple(range(0, N_SPEAKERS - LANES + 1, LANES)) + (
    N_SPEAKERS - LANES,
)

_mesh = plsc.VectorSubcoreMesh(
    core_axis_name="c",
    subcore_axis_name="s",
    num_cores=NUM_CORES,
    num_subcores=NUM_SUBCORES,
)


@functools.partial(
    pl.kernel,
    out_type=jax.ShapeDtypeStruct((BATCH, N_SPEAKERS), jnp.float32),
    mesh=_mesh,
    scratch_types=[
        pltpu.VMEM((ROWS_PER_WORKER,), jnp.int32),
        pltpu.VMEM((CHUNK_ROWS, N_SPEAKERS), jnp.float32),
        pltpu.VMEM((CHUNK_ROWS, N_SPEAKERS), jnp.float32),
        pltpu.SemaphoreType.DMA,
        pltpu.SemaphoreType.DMA,
    ],
    compiler_params=pltpu.CompilerParams(needs_layout_passes=False),
)
def _onehot_sc(ids_hbm, out_hbm, ids_v, buf0, buf1, sem0, sem1):
    wid = lax.axis_index("s") * NUM_CORES + lax.axis_index("c")
    row_base = wid * ROWS_PER_WORKER

    pltpu.sync_copy(ids_hbm.at[pl.ds(row_base, ROWS_PER_WORKER)], ids_v)

    zeros = jnp.zeros((LANES,), jnp.float32)
    ones = jnp.ones((LANES,), jnp.float32)
    lane_iota = lax.iota(jnp.int32, LANES)

    def zero_body(r, carry):
        for off in _COL_OFFSETS:
            buf0[r, pl.ds(off, LANES)] = zeros
            buf1[r, pl.ds(off, LANES)] = zeros
        return carry

    lax.fori_loop(0, CHUNK_ROWS, zero_body, 0)

    bufs = (buf0, buf1)
    sems = (sem0, sem1)
    copies = [None, None]
    prev_idx = [None, None]

    for c in range(NUM_CHUNKS):
        b = c % 2
        buf = bufs[b]
        if copies[b] is not None:
            copies[b].wait()
            for rows, cols in prev_idx[b]:
                plsc.store_scatter(buf, [rows, cols], zeros)
        idxs = []
        for g in range(GROUPS):
            cols = ids_v[pl.ds(c * CHUNK_ROWS + g * LANES, LANES)]
            rows = lane_iota + g * LANES
            plsc.store_scatter(buf, [rows, cols], ones)
            idxs.append((rows, cols))
        prev_idx[b] = idxs
        dst = out_hbm.at[pl.ds(row_base + c * CHUNK_ROWS, CHUNK_ROWS)]
        copies[b] = pltpu.async_copy(buf, dst, sems[b])

    for b in range(2):
        if copies[b] is not None:
            copies[b].wait()


def kernel(style_id):
    return _onehot_sc(style_id)


# speedup vs baseline: 3.9505x; 2.2231x over previous
import functools

import jax
import jax.numpy as jnp
from jax import lax
from jax.experimental import pallas as pl
from jax.experimental.pallas import tpu as pltpu
from jax.experimental.pallas import tpu_sc as plsc

N_SPEAKERS = 1000
BATCH = 16384

NUM_CORES = 2
NUM_SUBCORES = 16
LANES = 16
NUM_WORKERS = NUM_CORES * NUM_SUBCORES
COLS_PER_WORKER = BATCH // NUM_WORKERS
CHUNK_COLS = 128
COL_CHUNKS = COLS_PER_WORKER // CHUNK_COLS
BAND_ROWS = 200
ROW_BANDS = N_SPEAKERS // BAND_ROWS
COL_GROUPS = CHUNK_COLS // LANES

_mesh = plsc.VectorSubcoreMesh(
    core_axis_name="c",
    subcore_axis_name="s",
    num_cores=NUM_CORES,
    num_subcores=NUM_SUBCORES,
)


@functools.partial(
    pl.kernel,
    out_type=jax.ShapeDtypeStruct((N_SPEAKERS, BATCH), jnp.float32),
    mesh=_mesh,
    scratch_types=[
        pltpu.VMEM((COLS_PER_WORKER,), jnp.int32),
        pltpu.VMEM((BAND_ROWS, CHUNK_COLS), jnp.float32),
        pltpu.VMEM((BAND_ROWS, CHUNK_COLS), jnp.float32),
        pltpu.SemaphoreType.DMA,
        pltpu.SemaphoreType.DMA,
    ],
    compiler_params=pltpu.CompilerParams(needs_layout_passes=False),
)
def _onehot_sc_t(ids_hbm, out_hbm, ids_v, buf0, buf1, sem0, sem1):
    wid = lax.axis_index("s") * NUM_CORES + lax.axis_index("c")
    col_base = wid * COLS_PER_WORKER

    pltpu.sync_copy(ids_hbm.at[pl.ds(col_base, COLS_PER_WORKER)], ids_v)

    zeros = jnp.zeros((LANES,), jnp.float32)
    ones = jnp.ones((LANES,), jnp.float32)
    lane_iota = lax.iota(jnp.int32, LANES)

    def zero_body(r, carry):
        for off in range(0, CHUNK_COLS, LANES):
            buf0[r, pl.ds(off, LANES)] = zeros
            buf1[r, pl.ds(off, LANES)] = zeros
        return carry

    lax.fori_loop(0, BAND_ROWS, zero_body, 0)

    bufs = (buf0, buf1)
    sems = (sem0, sem1)
    copies = [None, None]
    prev = [None, None]

    def scatter_slab(buf, cc, r0, value):
        for g in range(COL_GROUPS):
            ids16 = ids_v[pl.ds(cc * CHUNK_COLS + g * LANES, LANES)]
            rows = ids16 - r0
            mask = (ids16 >= r0) & (ids16 < r0 + BAND_ROWS)
            cols = lane_iota + g * LANES
            plsc.store_scatter(buf, [rows, cols], value, mask=mask)

    for i in range(COL_CHUNKS * ROW_BANDS):
        cc, rb = divmod(i, ROW_BANDS)
        r0 = rb * BAND_ROWS
        b = i % 2
        buf = bufs[b]
        if copies[b] is not None:
            copies[b].wait()
            pcc, pr0 = prev[b]
            scatter_slab(buf, pcc, pr0, zeros)
        scatter_slab(buf, cc, r0, ones)
        prev[b] = (cc, r0)
        dst = out_hbm.at[
            pl.ds(r0, BAND_ROWS),
            pl.ds(col_base + cc * CHUNK_COLS, CHUNK_COLS),
        ]
        copies[b] = pltpu.async_copy(buf, dst, sems[b])

    for b in range(2):
        if copies[b] is not None:
            copies[b].wait()


def kernel(style_id):
    return _onehot_sc_t(style_id).T


# speedup vs baseline: 4.0183x; 1.0172x over previous
import functools

import jax
import jax.numpy as jnp
from jax import lax
from jax.experimental import pallas as pl
from jax.experimental.pallas import tpu as pltpu
from jax.experimental.pallas import tpu_sc as plsc

N_SPEAKERS = 1000
BATCH = 16384

NUM_CORES = 2
NUM_SUBCORES = 16
LANES = 16
NUM_WORKERS = NUM_CORES * NUM_SUBCORES
COLS_PER_WORKER = BATCH // NUM_WORKERS
CHUNK_COLS = 128
COL_CHUNKS = COLS_PER_WORKER // CHUNK_COLS
BAND_ROWS = 200
ROW_BANDS = N_SPEAKERS // BAND_ROWS
COL_GROUPS = CHUNK_COLS // LANES

_mesh = plsc.VectorSubcoreMesh(
    core_axis_name="c",
    subcore_axis_name="s",
    num_cores=NUM_CORES,
    num_subcores=NUM_SUBCORES,
)


@functools.partial(
    pl.kernel,
    out_type=jax.ShapeDtypeStruct((N_SPEAKERS, BATCH), jnp.float32),
    mesh=_mesh,
    scratch_types=[
        pltpu.VMEM((COLS_PER_WORKER,), jnp.int32),
        pltpu.VMEM((BAND_ROWS, CHUNK_COLS), jnp.float32),
        pltpu.VMEM((BAND_ROWS, CHUNK_COLS), jnp.float32),
        pltpu.SemaphoreType.DMA,
        pltpu.SemaphoreType.DMA,
    ],
    compiler_params=pltpu.CompilerParams(
        needs_layout_passes=False,
        disable_bounds_checks=True,
        skip_device_barrier=True,
    ),
)
def _onehot_sc_t(ids_hbm, out_hbm, ids_v, buf0, buf1, sem0, sem1):
    wid = lax.axis_index("s") * NUM_CORES + lax.axis_index("c")
    col_base = wid * COLS_PER_WORKER

    pltpu.sync_copy(ids_hbm.at[pl.ds(col_base, COLS_PER_WORKER)], ids_v)

    zeros = jnp.zeros((LANES,), jnp.float32)
    ones = jnp.ones((LANES,), jnp.float32)
    lane_iota = lax.iota(jnp.int32, LANES)

    def make_zero_body(buf):
        def zero_body(r, carry):
            for off in range(0, CHUNK_COLS, LANES):
                buf[r, pl.ds(off, LANES)] = zeros
            return carry

        return zero_body

    bufs = (buf0, buf1)
    sems = (sem0, sem1)
    copies = [None, None]
    prev = [None, None]

    def scatter_slab(buf, cc, r0, value):
        for g in range(COL_GROUPS):
            ids16 = ids_v[pl.ds(cc * CHUNK_COLS + g * LANES, LANES)]
            rows = ids16 - r0
            mask = (ids16 >= r0) & (ids16 < r0 + BAND_ROWS)
            cols = lane_iota + g * LANES
            plsc.store_scatter(buf, [rows, cols], value, mask=mask)

    for i in range(COL_CHUNKS * ROW_BANDS):
        cc, rb = divmod(i, ROW_BANDS)
        r0 = rb * BAND_ROWS
        b = i % 2
        buf = bufs[b]
        if i < 2:
            lax.fori_loop(0, BAND_ROWS, make_zero_body(buf), 0)
        if copies[b] is not None:
            copies[b].wait()
            pcc, pr0 = prev[b]
            scatter_slab(buf, pcc, pr0, zeros)
        scatter_slab(buf, cc, r0, ones)
        prev[b] = (cc, r0)
        dst = out_hbm.at[
            pl.ds(r0, BAND_ROWS),
            pl.ds(col_base + cc * CHUNK_COLS, CHUNK_COLS),
        ]
        copies[b] = pltpu.async_copy(buf, dst, sems[b])

    for b in range(2):
        if copies[b] is not None:
            copies[b].wait()


def kernel(style_id):
    return _onehot_sc_t(style_id).T


# speedup vs baseline: 4.0934x; 1.0187x over previous
import functools

import jax
import jax.numpy as jnp
from jax import lax
from jax.experimental import pallas as pl
from jax.experimental.pallas import tpu as pltpu
from jax.experimental.pallas import tpu_sc as plsc

N_SPEAKERS = 1000
BATCH = 16384

NUM_CORES = 2
NUM_SUBCORES = 16
LANES = 16
NUM_WORKERS = NUM_CORES * NUM_SUBCORES
COLS_PER_WORKER = BATCH // NUM_WORKERS
CHUNK_COLS = 128
COL_CHUNKS = COLS_PER_WORKER // CHUNK_COLS
BANDS = (504, 496)
BAND_STARTS = (0, 504)
BUF_ROWS = max(BANDS)
COL_GROUPS = CHUNK_COLS // LANES

_mesh = plsc.VectorSubcoreMesh(
    core_axis_name="c",
    subcore_axis_name="s",
    num_cores=NUM_CORES,
    num_subcores=NUM_SUBCORES,
)


@functools.partial(
    pl.kernel,
    out_type=jax.ShapeDtypeStruct((N_SPEAKERS, BATCH), jnp.float32),
    mesh=_mesh,
    scratch_types=[
        pltpu.VMEM((COLS_PER_WORKER,), jnp.int32),
        pltpu.VMEM((BUF_ROWS, CHUNK_COLS), jnp.float32),
        pltpu.VMEM((BUF_ROWS, CHUNK_COLS), jnp.float32),
        pltpu.SemaphoreType.DMA,
        pltpu.SemaphoreType.DMA,
    ],
    compiler_params=pltpu.CompilerParams(
        needs_layout_passes=False,
        disable_bounds_checks=True,
        skip_device_barrier=True,
    ),
)
def _onehot_sc_t(ids_hbm, out_hbm, ids_v, buf0, buf1, sem0, sem1):
    wid = lax.axis_index("s") * NUM_CORES + lax.axis_index("c")
    col_base = wid * COLS_PER_WORKER

    pltpu.sync_copy(ids_hbm.at[pl.ds(col_base, COLS_PER_WORKER)], ids_v)

    zeros = jnp.zeros((LANES,), jnp.float32)
    ones = jnp.ones((LANES,), jnp.float32)
    lane_iota = lax.iota(jnp.int32, LANES)

    def make_zero_body(buf):
        def zero_body(r, carry):
            for off in range(0, CHUNK_COLS, LANES):
                buf[r, pl.ds(off, LANES)] = zeros
            return carry

        return zero_body

    bufs = (buf0, buf1)
    sems = (sem0, sem1)
    copies = [None, None]
    prev = [None, None]

    def scatter_slab(buf, cc, rb, value):
        r0 = BAND_STARTS[rb]
        h = BANDS[rb]
        for g in range(COL_GROUPS):
            ids16 = ids_v[pl.ds(cc * CHUNK_COLS + g * LANES, LANES)]
            rows = ids16 - r0
            mask = (ids16 >= r0) & (ids16 < r0 + h)
            cols = lane_iota + g * LANES
            plsc.store_scatter(buf, [rows, cols], value, mask=mask)

    for i in range(COL_CHUNKS * len(BANDS)):
        cc, rb = divmod(i, len(BANDS))
        b = i % 2
        buf = bufs[b]
        if i < 2:
            lax.fori_loop(0, BUF_ROWS, make_zero_body(buf), 0)
        if copies[b] is not None:
            copies[b].wait()
            pcc, prb = prev[b]
            scatter_slab(buf, pcc, prb, zeros)
        scatter_slab(buf, cc, rb, ones)
        prev[b] = (cc, rb)
        h = BANDS[rb]
        dst = out_hbm.at[
            pl.ds(BAND_STARTS[rb], h),
            pl.ds(col_base + cc * CHUNK_COLS, CHUNK_COLS),
        ]
        copies[b] = pltpu.async_copy(buf.at[pl.ds(0, h)], dst, sems[b])

    for b in range(2):
        if copies[b] is not None:
            copies[b].wait()


def kernel(style_id):
    return _onehot_sc_t(style_id).T
